# baseline (speedup 1.0000x reference)
import functools
import math

import jax
import jax.numpy as jnp
from jax import lax
from jax.experimental import pallas as pl
from jax.experimental.pallas import tpu as pltpu
from jax.experimental.pallas import tpu_sc as plsc

D_MODEL = 768
SCALE = math.sqrt(float(D_MODEL))
LANES = 16
SLICES_PER_ROW = D_MODEL // LANES
CH = 16
RING = 4
LOOK = RING - 1


def _make_emb_kernel(B: int, D: int, NC: int, NS: int):
    NW = NC * NS
    b_per_w = B // NW
    n_chunks = b_per_w // CH
    n_groups = n_chunks // RING
    mesh = plsc.VectorSubcoreMesh(core_axis_name="c", subcore_axis_name="s")

    @functools.partial(
        pl.kernel,
        mesh=mesh,
        out_type=jax.ShapeDtypeStruct((B, D), jnp.float32),
        scratch_types=[
            pltpu.VMEM((b_per_w,), jnp.int32),
            pltpu.VMEM((RING, CH, D), jnp.float32),
            pltpu.VMEM((RING, CH, D), jnp.float32),
        ]
        + [pltpu.SemaphoreType.DMA] * (2 * RING),
    )
    def emb(idx_hbm, table_hbm, out_hbm, idx_v, rows_g, rows_o, *sems):
        sem_g = sems[:RING]
        sem_o = sems[RING:]
        wid = lax.axis_index("s") * NC + lax.axis_index("c")
        base = wid * b_per_w
        pltpu.sync_copy(idx_hbm.at[pl.ds(base, b_per_w)], idx_v)

        def start_g(c, b):
            return pltpu.async_copy(
                table_hbm.at[idx_v.at[pl.ds(c * CH, CH)]], rows_g.at[b], sem_g[b]
            )

        def wait_g(c, b):
            pltpu.make_async_copy(
                table_hbm.at[idx_v.at[pl.ds(c * CH, CH)]], rows_g.at[b], sem_g[b]
            ).wait()

        def start_o(c, b):
            return pltpu.async_copy(
                rows_o.at[b], out_hbm.at[pl.ds(base + c * CH, CH)], sem_o[b]
            )

        def wait_o(c, b):
            pltpu.make_async_copy(
                rows_o.at[b], out_hbm.at[pl.ds(base + c * CH, CH)], sem_o[b]
            ).wait()

        def scale(gb, ob):
            @plsc.parallel_loop(0, CH)
            def row_body(r):
                for s in range(SLICES_PER_ROW):
                    sl = pl.ds(s * LANES, LANES)
                    rows_o[ob, r, sl] = rows_g[gb, r, sl] * SCALE

        for c in range(LOOK):
            start_g(c, c)

        for b in range(RING):
            c = b
            start_g(c + LOOK, (c + LOOK) % RING)
            wait_g(c, b)
            scale(b, b)
            start_o(c, b)

        def group_body(p, _):
            for b in range(RING):
                c = p * RING + b
                start_g(c + LOOK, (b + LOOK) % RING)
                wait_g(c, b)
                wait_o(c - RING, b)
                scale(b, b)
                start_o(c, b)
            return 0

        lax.fori_loop(1, n_groups - 1, group_body, 0)

        for b in range(RING):
            c = (n_groups - 1) * RING + b
            if c + LOOK < n_chunks:
                start_g(c + LOOK, (b + LOOK) % RING)
            wait_g(c, b)
            wait_o(c - RING, b)
            scale(b, b)
            start_o(c, b)
        for b in range(RING):
            wait_o(n_chunks - RING + b, b)

    return emb


@jax.jit
def kernel(x, table):
    B0, S = x.shape
    V, D = table.shape
    idx = x.reshape(-1).astype(jnp.int32)
    info = plsc.get_sparse_core_info()
    emb = _make_emb_kernel(B0 * S, D, info.num_cores, info.num_subcores)
    out = emb(idx, table)
    return out.reshape(B0, S, D)


# speedup vs baseline: 1.0214x; 1.0214x over previous
import functools
import math

import jax
import jax.numpy as jnp
from jax import lax
from jax.experimental import pallas as pl
from jax.experimental.pallas import tpu as pltpu
from jax.experimental.pallas import tpu_sc as plsc

D_MODEL = 768
SCALE = math.sqrt(float(D_MODEL))
LANES = 16
SLICES_PER_ROW = D_MODEL // LANES
CH = 32
GR = 3
OR = 2
PERIOD = 6


def _make_emb_kernel(B: int, D: int, NC: int, NS: int):
    NW = NC * NS
    b_per_w = B // NW
    n_chunks = b_per_w // CH
    mesh = plsc.VectorSubcoreMesh(core_axis_name="c", subcore_axis_name="s")

    @functools.partial(
        pl.kernel,
        mesh=mesh,
        out_type=jax.ShapeDtypeStruct((B, D), jnp.float32),
        scratch_types=[
            pltpu.VMEM((b_per_w,), jnp.int32),
            pltpu.VMEM((GR, CH, D), jnp.float32),
            pltpu.VMEM((OR, CH, D), jnp.float32),
        ]
        + [pltpu.SemaphoreType.DMA] * (GR + OR),
    )
    def emb(idx_hbm, table_hbm, out_hbm, idx_v, rows_g, rows_o, *sems):
        sem_g = sems[:GR]
        sem_o = sems[GR:]
        wid = lax.axis_index("s") * NC + lax.axis_index("c")
        base = wid * b_per_w
        pltpu.sync_copy(idx_hbm.at[pl.ds(base, b_per_w)], idx_v)

        def start_g(c, b):
            return pltpu.async_copy(
                table_hbm.at[idx_v.at[pl.ds(c * CH, CH)]], rows_g.at[b], sem_g[b]
            )

        def wait_g(c, b):
            pltpu.make_async_copy(
                table_hbm.at[idx_v.at[pl.ds(c * CH, CH)]], rows_g.at[b], sem_g[b]
            ).wait()

        def start_o(c, b):
            return pltpu.async_copy(
                rows_o.at[b], out_hbm.at[pl.ds(base + c * CH, CH)], sem_o[b]
            )

        def wait_o(c, b):
            pltpu.make_async_copy(
                rows_o.at[b], out_hbm.at[pl.ds(base + c * CH, CH)], sem_o[b]
            ).wait()

        def scale(gb, ob):
            @plsc.parallel_loop(0, CH)
            def row_body(r):
                for s in range(SLICES_PER_ROW):
                    sl = pl.ds(s * LANES, LANES)
                    rows_o[ob, r, sl] = rows_g[gb, r, sl] * SCALE

        start_g(0, 0)
        start_g(1, 1)
        for c in range(PERIOD):
            start_g(c + 2, (c + 2) % GR)
            wait_g(c, c % GR)
            if c - OR >= 0:
                wait_o(c - OR, (c - OR) % OR)
            scale(c % GR, c % OR)
            start_o(c, c % OR)

        def group_body(p, _):
            for j in range(PERIOD):
                c = p * PERIOD + j
                start_g(c + 2, (j + 2) % GR)
                wait_g(c, j % GR)
                wait_o(c - OR, j % OR)
                scale(j % GR, j % OR)
                start_o(c, j % OR)
            return 0

        n_interior = (n_chunks - 2 - PERIOD) // PERIOD
        lax.fori_loop(1, 1 + n_interior, group_body, 0)

        for c in range(PERIOD + n_interior * PERIOD, n_chunks):
            if c + 2 < n_chunks:
                start_g(c + 2, (c + 2) % GR)
            wait_g(c, c % GR)
            wait_o(c - OR, (c - OR) % OR)
            scale(c % GR, c % OR)
            start_o(c, c % OR)
        for c in range(n_chunks - OR, n_chunks):
            wait_o(c, c % OR)

    return emb


@jax.jit
def kernel(x, table):
    B0, S = x.shape
    V, D = table.shape
    idx = x.reshape(-1).astype(jnp.int32)
    info = plsc.get_sparse_core_info()
    emb = _make_emb_kernel(B0 * S, D, info.num_cores, info.num_subcores)
    out = emb(idx, table)
    return out.reshape(B0, S, D)


# speedup vs baseline: 1.0397x; 1.0179x over previous
import functools
import math

import jax
import jax.numpy as jnp
from jax import lax
from jax.experimental import pallas as pl
from jax.experimental.pallas import tpu as pltpu
from jax.experimental.pallas import tpu_sc as plsc

D_MODEL = 768
SCALE = math.sqrt(float(D_MODEL))
LANES = 16
SLICES_PER_ROW = D_MODEL // LANES
CH = 32
RING = 4
LOOK = RING // 2


def _make_emb_kernel(B: int, D: int, NC: int, NS: int):
    NW = NC * NS
    b_per_w = B // NW
    n_chunks = b_per_w // CH
    n_groups = n_chunks // RING
    mesh = plsc.VectorSubcoreMesh(core_axis_name="c", subcore_axis_name="s")

    @functools.partial(
        pl.kernel,
        mesh=mesh,
        out_type=jax.ShapeDtypeStruct((B, D), jnp.float32),
        scratch_types=[
            pltpu.VMEM((b_per_w,), jnp.int32),
            pltpu.VMEM((RING, CH, D), jnp.float32),
        ]
        + [pltpu.SemaphoreType.DMA] * (2 * RING),
    )
    def emb(idx_hbm, table_hbm, out_hbm, idx_v, rows_v, *sems):
        sem_g = sems[:RING]
        sem_o = sems[RING:]
        wid = lax.axis_index("s") * NC + lax.axis_index("c")
        base = wid * b_per_w
        pltpu.sync_copy(idx_hbm.at[pl.ds(base, b_per_w)], idx_v)

        def start_g(c, b):
            return pltpu.async_copy(
                table_hbm.at[idx_v.at[pl.ds(c * CH, CH)]], rows_v.at[b], sem_g[b]
            )

        def wait_g(c, b):
            pltpu.make_async_copy(
                table_hbm.at[idx_v.at[pl.ds(c * CH, CH)]], rows_v.at[b], sem_g[b]
            ).wait()

        def start_o(c, b):
            return pltpu.async_copy(
                rows_v.at[b], out_hbm.at[pl.ds(base + c * CH, CH)], sem_o[b]
            )

        def wait_o(c, b):
            pltpu.make_async_copy(
                rows_v.at[b], out_hbm.at[pl.ds(base + c * CH, CH)], sem_o[b]
            ).wait()

        def scale(b):
            @plsc.parallel_loop(0, CH, unroll=2)
            def row_body(r):
                for s in range(SLICES_PER_ROW):
                    sl = pl.ds(s * LANES, LANES)
                    rows_v[b, r, sl] = rows_v[b, r, sl] * SCALE

        for c in range(LOOK):
            start_g(c, c % RING)
        for b in range(RING):
            c = b
            wait_g(c, b)
            scale(b)
            start_o(c, b)
            if c - LOOK >= 0:
                wait_o(c - LOOK, (c - LOOK) % RING)
            start_g(c + LOOK, (c + LOOK) % RING)

        def group_body(p, _):
            for b in range(RING):
                c = p * RING + b
                wait_g(c, b)
                scale(b)
                start_o(c, b)
                wait_o(c - LOOK, (b - LOOK) % RING)
                start_g(c + LOOK, (b + LOOK) % RING)
            return 0

        lax.fori_loop(1, n_groups - 1, group_body, 0)

        for b in range(RING):
            c = (n_groups - 1) * RING + b
            wait_g(c, b)
            scale(b)
            start_o(c, b)
            wait_o(c - LOOK, (b - LOOK) % RING)
            if c + LOOK < n_chunks:
                start_g(c + LOOK, (b + LOOK) % RING)
        for c in range(n_chunks - LOOK, n_chunks):
            wait_o(c, c % RING)

    return emb


@jax.jit
def kernel(x, table):
    B0, S = x.shape
    V, D = table.shape
    idx = x.reshape(-1).astype(jnp.int32)
    info = plsc.get_sparse_core_info()
    emb = _make_emb_kernel(B0 * S, D, info.num_cores, info.num_subcores)
    out = emb(idx, table)
    return out.reshape(B0, S, D)


# speedup vs baseline: 1.0874x; 1.0458x over previous
import functools
import math

import jax
import jax.numpy as jnp
from jax import lax
from jax.experimental import pallas as pl
from jax.experimental.pallas import tpu as pltpu
from jax.experimental.pallas import tpu_sc as plsc

D_MODEL = 768
SCALE = math.sqrt(float(D_MODEL))
LANES = 16
SLICES_PER_ROW = D_MODEL // LANES
CH = 32
RING = 4
LOOK = RING // 2


def _make_emb_kernel(B: int, D: int, NC: int, NS: int):
    NW = NC * NS
    b_per_w = B // NW
    n_chunks = b_per_w // CH
    n_groups = n_chunks // RING
    mesh = plsc.VectorSubcoreMesh(core_axis_name="c", subcore_axis_name="s")

    @functools.partial(
        pl.kernel,
        mesh=mesh,
        out_type=jax.ShapeDtypeStruct((B, D), jnp.float32),
        scratch_types=[
            pltpu.VMEM((b_per_w,), jnp.int32),
            pltpu.VMEM((RING, CH, D), jnp.float32),
        ]
        + [pltpu.SemaphoreType.DMA] * (2 * RING),
    )
    def emb(idx_hbm, table_hbm, out_hbm, idx_v, rows_v, *sems):
        sem_g = sems[:RING]
        sem_o = sems[RING:]
        wid = lax.axis_index("s") * NC + lax.axis_index("c")
        base = wid * b_per_w
        pltpu.sync_copy(idx_hbm.at[pl.ds(base, b_per_w)], idx_v)

        def start_g(c, b):
            return pltpu.async_copy(
                table_hbm.at[idx_v.at[pl.ds(c * CH, CH)]], rows_v.at[b], sem_g[b]
            )

        def wait_g(c, b):
            pltpu.make_async_copy(
                table_hbm.at[idx_v.at[pl.ds(c * CH, CH)]], rows_v.at[b], sem_g[b]
            ).wait()

        def start_o(c, b):
            return pltpu.async_copy(
                rows_v.at[b], out_hbm.at[pl.ds(base + c * CH, CH)], sem_o[b]
            )

        def wait_o(c, b):
            pltpu.make_async_copy(
                rows_v.at[b], out_hbm.at[pl.ds(base + c * CH, CH)], sem_o[b]
            ).wait()

        def scale(b):
            @plsc.parallel_loop(0, CH)
            def row_body(r):
                for s in range(SLICES_PER_ROW):
                    sl = pl.ds(s * LANES, LANES)
                    rows_v[b, r, sl] = rows_v[b, r, sl] * SCALE

        for c in range(LOOK):
            start_g(c, c % RING)
        for b in range(RING):
            c = b
            wait_g(c, b)
            if c - LOOK >= 0:
                wait_o(c - LOOK, (c - LOOK) % RING)
            start_g(c + LOOK, (c + LOOK) % RING)
            scale(b)
            start_o(c, b)

        def group_body(p, _):
            for b in range(RING):
                c = p * RING + b
                wait_g(c, b)
                wait_o(c - LOOK, (b - LOOK) % RING)
                start_g(c + LOOK, (b + LOOK) % RING)
                scale(b)
                start_o(c, b)
            return 0

        lax.fori_loop(1, n_groups - 1, group_body, 0)

        for b in range(RING):
            c = (n_groups - 1) * RING + b
            wait_g(c, b)
            wait_o(c - LOOK, (b - LOOK) % RING)
            if c + LOOK < n_chunks:
                start_g(c + LOOK, (b + LOOK) % RING)
            scale(b)
            start_o(c, b)
        for c in range(n_chunks - LOOK, n_chunks):
            wait_o(c, c % RING)

    return emb


@jax.jit
def kernel(x, table):
    B0, S = x.shape
    V, D = table.shape
    idx = x.reshape(-1).astype(jnp.int32)
    info = plsc.get_sparse_core_info()
    emb = _make_emb_kernel(B0 * S, D, info.num_cores, info.num_subcores)
    out = emb(idx, table)
    return out.reshape(B0, S, D)
